# baseline (speedup 1.0000x reference)
import jax
import jax.numpy as jnp
from jax import lax
from jax.experimental import pallas as pl
from jax.experimental.pallas import tpu as pltpu
from jax.experimental.pallas import tpu_sc as plsc

N = 10000
D = 128
E = 320000
NC = 2
NS = 16
NW = NC * NS
CHUNK = 64
CPT = 162
EPT = CPT * CHUNK
EPAD = NW * EPT
NPAD = 10240
RPT = NPAD // NS
DUMMY = NPAD - 1

_mesh = plsc.VectorSubcoreMesh(
    core_axis_name="c", subcore_axis_name="s", num_cores=NC, num_subcores=NS
)


def _scat_body(src_hbm, dst_hbm, h_hbm, zeros_hbm, out_hbm,
               src_v, dst_v, buf0, buf1, acc, sem0, sem1, sem2, sem3):
    c = lax.axis_index("c")
    s = lax.axis_index("s")
    wid = c * NS + s
    pltpu.sync_copy(zeros_hbm.at[pl.ds(s * RPT, RPT)], acc.at[pl.ds(s * RPT, RPT)])
    plsc.subcore_barrier()

    half = CPT // 2
    for t in range(2):
        pltpu.sync_copy(src_hbm.at[wid * 2 + t], src_v)
        pltpu.sync_copy(dst_hbm.at[wid * 2 + t], dst_v)
        pltpu.async_copy(h_hbm.at[src_v.at[0]], buf0, sem0)
        pltpu.async_copy(h_hbm.at[src_v.at[1]], buf1, sem1)

        def body(i, carry):
            j = 2 * i
            pltpu.make_async_copy(h_hbm.at[src_v.at[j]], buf0, sem0).wait()
            pltpu.async_copy(buf0, acc.at[dst_v.at[j]], sem2, add=True)
            pltpu.make_async_copy(h_hbm.at[src_v.at[j + 1]], buf1, sem1).wait()
            pltpu.async_copy(buf1, acc.at[dst_v.at[j + 1]], sem3, add=True)
            pltpu.make_async_copy(buf0, acc.at[dst_v.at[j]], sem2).wait()
            pltpu.async_copy(h_hbm.at[src_v.at[j + 2]], buf0, sem0)
            pltpu.make_async_copy(buf1, acc.at[dst_v.at[j + 1]], sem3).wait()

            @pl.when(j + 3 < half)
            def _():
                pltpu.async_copy(h_hbm.at[src_v.at[j + 3]], buf1, sem1)

            return carry

        lax.fori_loop(0, (half - 1) // 2, body, 0)
        pltpu.make_async_copy(h_hbm.at[src_v.at[half - 1]], buf0, sem0).wait()
        pltpu.async_copy(buf0, acc.at[dst_v.at[half - 1]], sem2, add=True)
        pltpu.make_async_copy(buf0, acc.at[dst_v.at[half - 1]], sem2).wait()
    plsc.subcore_barrier()
    pltpu.sync_copy(acc.at[pl.ds(s * RPT, RPT)], out_hbm.at[c, pl.ds(s * RPT, RPT)])


_scat_call = pl.kernel(
    _scat_body,
    out_type=jax.ShapeDtypeStruct((NC, NPAD, D), jnp.float32),
    mesh=_mesh,
    scratch_types=[
        pltpu.VMEM((CPT // 2, CHUNK), jnp.int32),
        pltpu.VMEM((CPT // 2, CHUNK), jnp.int32),
        pltpu.VMEM((CHUNK, D), jnp.float32),
        pltpu.VMEM((CHUNK, D), jnp.float32),
        pltpu.VMEM_SHARED((NPAD, D), jnp.float32),
        pltpu.SemaphoreType.DMA,
        pltpu.SemaphoreType.DMA,
        pltpu.SemaphoreType.DMA,
        pltpu.SemaphoreType.DMA,
    ],
)


BM = 1024


def _dinv(d0_ref, d1_ref):
    deg = d0_ref[:, 0:1] + d1_ref[:, 0:1]
    return jnp.where(deg > 0, lax.rsqrt(deg), 0.0)


def _mm_body(x_ref, w_ref, d0_ref, d1_ref, o_ref):
    dinv = _dinv(d0_ref, d1_ref)
    o_ref[...] = jnp.dot(x_ref[...], w_ref[...],
                         preferred_element_type=jnp.float32) * dinv


_mm_call = pl.pallas_call(
    _mm_body,
    grid=(NPAD // BM,),
    in_specs=[
        pl.BlockSpec((BM, D), lambda i: (i, 0)),
        pl.BlockSpec((D, D), lambda i: (0, 0)),
        pl.BlockSpec((BM, 8), lambda i: (i, 0)),
        pl.BlockSpec((BM, 8), lambda i: (i, 0)),
    ],
    out_specs=pl.BlockSpec((BM, D), lambda i: (i, 0)),
    out_shape=jax.ShapeDtypeStruct((NPAD, D), jnp.float32),
)


def _comb_mm_body(p0_ref, p1_ref, d0_ref, d1_ref, b_ref, w_ref, emb_ref, h2_ref):
    dinv = _dinv(d0_ref, d1_ref)
    emb = (p0_ref[...] + p1_ref[...]) * dinv + b_ref[...]
    emb_ref[...] = emb
    h2_ref[...] = jnp.dot(emb, w_ref[...],
                          preferred_element_type=jnp.float32) * dinv


_comb_mm_call = pl.pallas_call(
    _comb_mm_body,
    grid=(NPAD // BM,),
    in_specs=[
        pl.BlockSpec((BM, D), lambda i: (i, 0)),
        pl.BlockSpec((BM, D), lambda i: (i, 0)),
        pl.BlockSpec((BM, 8), lambda i: (i, 0)),
        pl.BlockSpec((BM, 8), lambda i: (i, 0)),
        pl.BlockSpec((1, D), lambda i: (0, 0)),
        pl.BlockSpec((D, D), lambda i: (0, 0)),
    ],
    out_specs=[
        pl.BlockSpec((BM, D), lambda i: (i, 0)),
        pl.BlockSpec((BM, D), lambda i: (i, 0)),
    ],
    out_shape=[
        jax.ShapeDtypeStruct((NPAD, D), jnp.float32),
        jax.ShapeDtypeStruct((NPAD, D), jnp.float32),
    ],
)


def _comb_body(p0_ref, p1_ref, d0_ref, d1_ref, b_ref, o_ref):
    dinv = _dinv(d0_ref, d1_ref)
    o_ref[...] = (p0_ref[...] + p1_ref[...]) * dinv + b_ref[...]


_comb_call = pl.pallas_call(
    _comb_body,
    grid=(NPAD // BM,),
    in_specs=[
        pl.BlockSpec((BM, D), lambda i: (i, 0)),
        pl.BlockSpec((BM, D), lambda i: (i, 0)),
        pl.BlockSpec((BM, 8), lambda i: (i, 0)),
        pl.BlockSpec((BM, 8), lambda i: (i, 0)),
        pl.BlockSpec((1, D), lambda i: (0, 0)),
    ],
    out_specs=pl.BlockSpec((BM, D), lambda i: (i, 0)),
    out_shape=jax.ShapeDtypeStruct((NPAD, D), jnp.float32),
)


def kernel(x, edge_index, W_enc, b_enc, W_dec, b_dec):
    src = edge_index[0]
    dst = edge_index[1]
    loop = jnp.arange(N, dtype=edge_index.dtype)
    pad = jnp.full((EPAD - E - N,), DUMMY, dtype=edge_index.dtype)
    src_a = jnp.concatenate([src, loop, pad]).reshape(NW, CPT, CHUNK)
    dst_a = jnp.concatenate([dst, loop, pad]).reshape(NW, CPT, CHUNK)
    xp = jnp.concatenate([x, jnp.zeros((NPAD - N, D), x.dtype)])
    z128 = jnp.zeros((NPAD, D), jnp.float32)
    ones_mat = jnp.ones((NPAD, D), jnp.float32)

    src_h = src_a.reshape(NW * 2, CPT // 2, CHUNK)
    dst_h = dst_a.reshape(NW * 2, CPT // 2, CHUNK)
    degp = _scat_call(jnp.zeros_like(src_h), dst_h, ones_mat, z128)
    d0, d1 = degp[0, :, :8], degp[1, :, :8]
    h1 = _mm_call(xp, W_enc, d0, d1)
    p = _scat_call(src_h, dst_h, h1, z128)
    emb, h2 = _comb_mm_call(p[0], p[1], d0, d1, b_enc.reshape(1, D), W_dec)
    p2 = _scat_call(src_h, dst_h, h2, z128)
    rec = _comb_call(p2[0], p2[1], d0, d1, b_dec.reshape(1, D))
    return emb[:N], rec[:N]


# speedup vs baseline: 18.6174x; 18.6174x over previous
import jax
import jax.numpy as jnp
from jax import lax
from jax.experimental import pallas as pl
from jax.experimental.pallas import tpu as pltpu
from jax.experimental.pallas import tpu_sc as plsc

N = 10000
D = 128
E = 320000
NC = 2
NS = 16
NW = NC * NS
CHUNK = 64
CPT = 162
EPT = CPT * CHUNK
EPAD = NW * EPT
NPAD = 10240
RPT = NPAD // NS
DUMMY = NPAD - 1

_mesh = plsc.VectorSubcoreMesh(
    core_axis_name="c", subcore_axis_name="s", num_cores=NC, num_subcores=NS
)


def _scat_body(src_hbm, dst_hbm, h_hbm, zeros_hbm, flag_hbm, out_hbm,
               src_v, dst_v, buf0, buf1, flag_v, acc, sem0, sem1, sem2, sem3):
    c = lax.axis_index("c")
    s = lax.axis_index("s")
    wid = c * NS + s
    pltpu.sync_copy(zeros_hbm.at[pl.ds(s * RPT, RPT)], acc.at[pl.ds(s * RPT, RPT)])
    pltpu.sync_copy(flag_hbm, flag_v)
    gath = flag_v[...][0] > 0

    @pl.when(jnp.logical_not(gath))
    def _():
        pltpu.sync_copy(h_hbm.at[pl.ds(0, CHUNK)], buf0)
        pltpu.sync_copy(h_hbm.at[pl.ds(0, CHUNK)], buf1)

    plsc.subcore_barrier()

    half = CPT // 2
    for t in range(2):
        pltpu.sync_copy(src_hbm.at[wid * 2 + t], src_v)
        pltpu.sync_copy(dst_hbm.at[wid * 2 + t], dst_v)
        @pl.when(gath)
        def _():
            pltpu.async_copy(h_hbm.at[src_v.at[0]], buf0, sem0)
            pltpu.async_copy(h_hbm.at[src_v.at[1]], buf1, sem1)

        def body(i, carry):
            j = 2 * i

            @pl.when(gath)
            def _():
                pltpu.make_async_copy(h_hbm.at[src_v.at[j]], buf0, sem0).wait()

            pltpu.async_copy(buf0, acc.at[dst_v.at[j]], sem2, add=True)

            @pl.when(gath)
            def _():
                pltpu.make_async_copy(h_hbm.at[src_v.at[j + 1]], buf1, sem1).wait()

            pltpu.async_copy(buf1, acc.at[dst_v.at[j + 1]], sem3, add=True)
            pltpu.make_async_copy(buf0, acc.at[dst_v.at[j]], sem2).wait()

            @pl.when(gath)
            def _():
                pltpu.async_copy(h_hbm.at[src_v.at[j + 2]], buf0, sem0)

            pltpu.make_async_copy(buf1, acc.at[dst_v.at[j + 1]], sem3).wait()

            @pl.when(jnp.logical_and(gath, j + 3 < half))
            def _():
                pltpu.async_copy(h_hbm.at[src_v.at[j + 3]], buf1, sem1)

            return carry

        lax.fori_loop(0, (half - 1) // 2, body, 0)

        @pl.when(gath)
        def _():
            pltpu.make_async_copy(h_hbm.at[src_v.at[half - 1]], buf0, sem0).wait()

        pltpu.async_copy(buf0, acc.at[dst_v.at[half - 1]], sem2, add=True)
        pltpu.make_async_copy(buf0, acc.at[dst_v.at[half - 1]], sem2).wait()
    plsc.subcore_barrier()
    pltpu.sync_copy(acc.at[pl.ds(s * RPT, RPT)], out_hbm.at[c, pl.ds(s * RPT, RPT)])


_scat_call = pl.kernel(
    _scat_body,
    out_type=jax.ShapeDtypeStruct((NC, NPAD, D), jnp.float32),
    mesh=_mesh,
    scratch_types=[
        pltpu.VMEM((CPT // 2, CHUNK), jnp.int32),
        pltpu.VMEM((CPT // 2, CHUNK), jnp.int32),
        pltpu.VMEM((CHUNK, D), jnp.float32),
        pltpu.VMEM((CHUNK, D), jnp.float32),
        pltpu.VMEM((16,), jnp.int32),
        pltpu.VMEM_SHARED((NPAD, D), jnp.float32),
        pltpu.SemaphoreType.DMA,
        pltpu.SemaphoreType.DMA,
        pltpu.SemaphoreType.DMA,
        pltpu.SemaphoreType.DMA,
    ],
)


BM = 1024


def _dinv(d0_ref, d1_ref):
    deg = d0_ref[:, 0:1] + d1_ref[:, 0:1]
    return jnp.where(deg > 0, lax.rsqrt(deg), 0.0)


def _mm_body(x_ref, w_ref, d0_ref, d1_ref, o_ref):
    dinv = _dinv(d0_ref, d1_ref)
    o_ref[...] = jnp.dot(x_ref[...], w_ref[...],
                         preferred_element_type=jnp.float32) * dinv


_mm_call = pl.pallas_call(
    _mm_body,
    grid=(NPAD // BM,),
    in_specs=[
        pl.BlockSpec((BM, D), lambda i: (i, 0)),
        pl.BlockSpec((D, D), lambda i: (0, 0)),
        pl.BlockSpec((BM, 8), lambda i: (i, 0)),
        pl.BlockSpec((BM, 8), lambda i: (i, 0)),
    ],
    out_specs=pl.BlockSpec((BM, D), lambda i: (i, 0)),
    out_shape=jax.ShapeDtypeStruct((NPAD, D), jnp.float32),
)


def _comb_mm_body(p0_ref, p1_ref, d0_ref, d1_ref, b_ref, w_ref, emb_ref, h2_ref):
    dinv = _dinv(d0_ref, d1_ref)
    emb = (p0_ref[...] + p1_ref[...]) * dinv + b_ref[...]
    emb_ref[...] = emb
    h2_ref[...] = jnp.dot(emb, w_ref[...],
                          preferred_element_type=jnp.float32) * dinv


_comb_mm_call = pl.pallas_call(
    _comb_mm_body,
    grid=(NPAD // BM,),
    in_specs=[
        pl.BlockSpec((BM, D), lambda i: (i, 0)),
        pl.BlockSpec((BM, D), lambda i: (i, 0)),
        pl.BlockSpec((BM, 8), lambda i: (i, 0)),
        pl.BlockSpec((BM, 8), lambda i: (i, 0)),
        pl.BlockSpec((1, D), lambda i: (0, 0)),
        pl.BlockSpec((D, D), lambda i: (0, 0)),
    ],
    out_specs=[
        pl.BlockSpec((BM, D), lambda i: (i, 0)),
        pl.BlockSpec((BM, D), lambda i: (i, 0)),
    ],
    out_shape=[
        jax.ShapeDtypeStruct((NPAD, D), jnp.float32),
        jax.ShapeDtypeStruct((NPAD, D), jnp.float32),
    ],
)


def _comb_body(p0_ref, p1_ref, d0_ref, d1_ref, b_ref, o_ref):
    dinv = _dinv(d0_ref, d1_ref)
    o_ref[...] = (p0_ref[...] + p1_ref[...]) * dinv + b_ref[...]


_comb_call = pl.pallas_call(
    _comb_body,
    grid=(NPAD // BM,),
    in_specs=[
        pl.BlockSpec((BM, D), lambda i: (i, 0)),
        pl.BlockSpec((BM, D), lambda i: (i, 0)),
        pl.BlockSpec((BM, 8), lambda i: (i, 0)),
        pl.BlockSpec((BM, 8), lambda i: (i, 0)),
        pl.BlockSpec((1, D), lambda i: (0, 0)),
    ],
    out_specs=pl.BlockSpec((BM, D), lambda i: (i, 0)),
    out_shape=jax.ShapeDtypeStruct((NPAD, D), jnp.float32),
)


def kernel(x, edge_index, W_enc, b_enc, W_dec, b_dec):
    src = edge_index[0]
    dst = edge_index[1]
    loop = jnp.arange(N, dtype=edge_index.dtype)
    pad = jnp.full((EPAD - E - N,), DUMMY, dtype=edge_index.dtype)
    src_a = jnp.concatenate([src, loop, pad]).reshape(NW, CPT, CHUNK)
    dst_a = jnp.concatenate([dst, loop, pad]).reshape(NW, CPT, CHUNK)
    xp = jnp.concatenate([x, jnp.zeros((NPAD - N, D), x.dtype)])
    z128 = jnp.zeros((NPAD, D), jnp.float32)
    ones_mat = jnp.ones((NPAD, D), jnp.float32)
    f0 = jnp.zeros((16,), jnp.int32)
    f1 = jnp.ones((16,), jnp.int32)

    src_h = src_a.reshape(NW * 2, CPT // 2, CHUNK)
    dst_h = dst_a.reshape(NW * 2, CPT // 2, CHUNK)
    degp = _scat_call(src_h, dst_h, ones_mat, z128, f0)
    d0, d1 = degp[0, :, :8], degp[1, :, :8]
    h1 = _mm_call(xp, W_enc, d0, d1)
    p = _scat_call(src_h, dst_h, h1, z128, f1)
    emb, h2 = _comb_mm_call(p[0], p[1], d0, d1, b_enc.reshape(1, D), W_dec)
    p2 = _scat_call(src_h, dst_h, h2, z128, f1)
    rec = _comb_call(p2[0], p2[1], d0, d1, b_dec.reshape(1, D))
    return emb[:N], rec[:N]


# speedup vs baseline: 23.5441x; 1.2646x over previous
import jax
import jax.numpy as jnp
from jax import lax
from jax.experimental import pallas as pl
from jax.experimental.pallas import tpu as pltpu
from jax.experimental.pallas import tpu_sc as plsc

N = 10000
D = 128
E = 320000
NC = 2
NS = 16
NW = NC * NS
CHUNK = 64
CPT = 162
EPT = CPT * CHUNK
EPAD = NW * EPT
NPAD = 10240
RPT = NPAD // NS
DUMMY = NPAD - 1

_mesh = plsc.VectorSubcoreMesh(
    core_axis_name="c", subcore_axis_name="s", num_cores=NC, num_subcores=NS
)


def _scat_body(src_hbm, dst_hbm, h_hbm, zeros_hbm, flag_hbm, out_hbm,
               src_v, dst_v, buf0, buf1, buf2, flag_v, acc,
               sem0, sem1, sem2, sem3, sem4, sem5):
    c = lax.axis_index("c")
    s = lax.axis_index("s")
    wid = c * NS + s
    pltpu.sync_copy(zeros_hbm.at[pl.ds(s * RPT, RPT)], acc.at[pl.ds(s * RPT, RPT)])
    pltpu.sync_copy(flag_hbm, flag_v)
    gath = flag_v[...][0] > 0

    @pl.when(jnp.logical_not(gath))
    def _():
        pltpu.sync_copy(h_hbm.at[pl.ds(0, CHUNK)], buf0)
        pltpu.sync_copy(h_hbm.at[pl.ds(0, CHUNK)], buf1)

    plsc.subcore_barrier()

    third = CPT // 3
    for t in range(3):
        pltpu.sync_copy(src_hbm.at[wid * 3 + t], src_v)
        pltpu.sync_copy(dst_hbm.at[wid * 3 + t], dst_v)

        @pl.when(gath)
        def _():
            pltpu.async_copy(h_hbm.at[src_v.at[0]], buf0, sem0)
            pltpu.async_copy(h_hbm.at[src_v.at[1]], buf1, sem1)
            pltpu.async_copy(h_hbm.at[src_v.at[2]], buf2, sem2)

        def body(i, carry):
            j = 3 * i
            for (jj, bb, gsem, csem) in ((j, buf0, sem0, sem3),
                                         (j + 1, buf1, sem1, sem4),
                                         (j + 2, buf2, sem2, sem5)):
                @pl.when(gath)
                def _():
                    pltpu.make_async_copy(h_hbm.at[src_v.at[jj]], bb, gsem).wait()

                pltpu.async_copy(bb, acc.at[dst_v.at[jj]], csem, add=True)
                pltpu.make_async_copy(bb, acc.at[dst_v.at[jj]], csem).wait()

                @pl.when(jnp.logical_and(gath, jj + 3 < third))
                def _():
                    pltpu.async_copy(h_hbm.at[src_v.at[jj + 3]], bb, gsem)

            return carry

        lax.fori_loop(0, third // 3, body, 0)
    plsc.subcore_barrier()
    pltpu.sync_copy(acc.at[pl.ds(s * RPT, RPT)], out_hbm.at[c, pl.ds(s * RPT, RPT)])


_scat_call = pl.kernel(
    _scat_body,
    out_type=jax.ShapeDtypeStruct((NC, NPAD, D), jnp.float32),
    mesh=_mesh,
    scratch_types=[
        pltpu.VMEM((CPT // 3, CHUNK), jnp.int32),
        pltpu.VMEM((CPT // 3, CHUNK), jnp.int32),
        pltpu.VMEM((CHUNK, D), jnp.float32),
        pltpu.VMEM((CHUNK, D), jnp.float32),
        pltpu.VMEM((CHUNK, D), jnp.float32),
        pltpu.VMEM((16,), jnp.int32),
        pltpu.VMEM_SHARED((NPAD, D), jnp.float32),
        pltpu.SemaphoreType.DMA,
        pltpu.SemaphoreType.DMA,
        pltpu.SemaphoreType.DMA,
        pltpu.SemaphoreType.DMA,
        pltpu.SemaphoreType.DMA,
        pltpu.SemaphoreType.DMA,
    ],
)


BM = 1024


def _dinv(d0_ref, d1_ref):
    deg = d0_ref[:, 0:1] + d1_ref[:, 0:1]
    return jnp.where(deg > 0, lax.rsqrt(deg), 0.0)


def _mm_body(x_ref, w_ref, d0_ref, d1_ref, o_ref):
    dinv = _dinv(d0_ref, d1_ref)
    o_ref[...] = jnp.dot(x_ref[...], w_ref[...],
                         preferred_element_type=jnp.float32) * dinv


_mm_call = pl.pallas_call(
    _mm_body,
    grid=(NPAD // BM,),
    in_specs=[
        pl.BlockSpec((BM, D), lambda i: (i, 0)),
        pl.BlockSpec((D, D), lambda i: (0, 0)),
        pl.BlockSpec((BM, 8), lambda i: (i, 0)),
        pl.BlockSpec((BM, 8), lambda i: (i, 0)),
    ],
    out_specs=pl.BlockSpec((BM, D), lambda i: (i, 0)),
    out_shape=jax.ShapeDtypeStruct((NPAD, D), jnp.float32),
)


def _comb_mm_body(p0_ref, p1_ref, d0_ref, d1_ref, b_ref, w_ref, emb_ref, h2_ref):
    dinv = _dinv(d0_ref, d1_ref)
    emb = (p0_ref[...] + p1_ref[...]) * dinv + b_ref[...]
    emb_ref[...] = emb
    h2_ref[...] = jnp.dot(emb, w_ref[...],
                          preferred_element_type=jnp.float32) * dinv


_comb_mm_call = pl.pallas_call(
    _comb_mm_body,
    grid=(NPAD // BM,),
    in_specs=[
        pl.BlockSpec((BM, D), lambda i: (i, 0)),
        pl.BlockSpec((BM, D), lambda i: (i, 0)),
        pl.BlockSpec((BM, 8), lambda i: (i, 0)),
        pl.BlockSpec((BM, 8), lambda i: (i, 0)),
        pl.BlockSpec((1, D), lambda i: (0, 0)),
        pl.BlockSpec((D, D), lambda i: (0, 0)),
    ],
    out_specs=[
        pl.BlockSpec((BM, D), lambda i: (i, 0)),
        pl.BlockSpec((BM, D), lambda i: (i, 0)),
    ],
    out_shape=[
        jax.ShapeDtypeStruct((NPAD, D), jnp.float32),
        jax.ShapeDtypeStruct((NPAD, D), jnp.float32),
    ],
)


def _comb_body(p0_ref, p1_ref, d0_ref, d1_ref, b_ref, o_ref):
    dinv = _dinv(d0_ref, d1_ref)
    o_ref[...] = (p0_ref[...] + p1_ref[...]) * dinv + b_ref[...]


_comb_call = pl.pallas_call(
    _comb_body,
    grid=(NPAD // BM,),
    in_specs=[
        pl.BlockSpec((BM, D), lambda i: (i, 0)),
        pl.BlockSpec((BM, D), lambda i: (i, 0)),
        pl.BlockSpec((BM, 8), lambda i: (i, 0)),
        pl.BlockSpec((BM, 8), lambda i: (i, 0)),
        pl.BlockSpec((1, D), lambda i: (0, 0)),
    ],
    out_specs=pl.BlockSpec((BM, D), lambda i: (i, 0)),
    out_shape=jax.ShapeDtypeStruct((NPAD, D), jnp.float32),
)


def kernel(x, edge_index, W_enc, b_enc, W_dec, b_dec):
    src = edge_index[0]
    dst = edge_index[1]
    loop = jnp.arange(N, dtype=edge_index.dtype)
    pad = jnp.full((EPAD - E - N,), DUMMY, dtype=edge_index.dtype)
    src_a = jnp.concatenate([src, loop, pad]).reshape(NW, CPT, CHUNK)
    dst_a = jnp.concatenate([dst, loop, pad]).reshape(NW, CPT, CHUNK)
    xp = jnp.concatenate([x, jnp.zeros((NPAD - N, D), x.dtype)])
    z128 = jnp.zeros((NPAD, D), jnp.float32)
    ones_mat = jnp.ones((NPAD, D), jnp.float32)
    f0 = jnp.zeros((16,), jnp.int32)
    f1 = jnp.ones((16,), jnp.int32)

    src_h = src_a.reshape(NW * 3, CPT // 3, CHUNK)
    dst_h = dst_a.reshape(NW * 3, CPT // 3, CHUNK)
    degp = _scat_call(src_h, dst_h, ones_mat, z128, f0)
    d0, d1 = degp[0, :, :8], degp[1, :, :8]
    h1 = _mm_call(xp, W_enc, d0, d1)
    p = _scat_call(src_h, dst_h, h1, z128, f1)
    emb, h2 = _comb_mm_call(p[0], p[1], d0, d1, b_enc.reshape(1, D), W_dec)
    p2 = _scat_call(src_h, dst_h, h2, z128, f1)
    rec = _comb_call(p2[0], p2[1], d0, d1, b_dec.reshape(1, D))
    return emb[:N], rec[:N]
